# Initial kernel scaffold; baseline (speedup 1.0000x reference)
import jax
import jax.numpy as jnp
from jax.experimental import pallas as pl


def kernel(condition, tables, W1, b1, W2, b2):
    raise NotImplementedError("write your pallas kernel here")


# speedup vs baseline: 7.5248x; 7.5248x over previous
import functools

import jax
import jax.numpy as jnp
from jax import lax
from jax.experimental import pallas as pl
from jax.experimental.pallas import tpu as pltpu
from jax.experimental.pallas import tpu_sc as plsc

N_FIELDS = 26
VOCAB = 100000
EMBED = 16
COND_DIM = N_FIELDS * EMBED
BATCH = 16384

TOTAL_ROWS = BATCH * N_FIELDS
NW = 32
ROWS_PER_W = TOTAL_ROWS // NW
CHUNK = 128
NCHUNK = ROWS_PER_W // CHUNK


def _make_sc_gather():
    mesh = plsc.VectorSubcoreMesh(core_axis_name="c", subcore_axis_name="s")

    @functools.partial(
        pl.kernel,
        mesh=mesh,
        out_type=jax.ShapeDtypeStruct((TOTAL_ROWS, EMBED), jnp.float32),
        scratch_types=[
            pltpu.VMEM((NCHUNK, CHUNK), jnp.int32),
            pltpu.VMEM((CHUNK, EMBED), jnp.float32),
            pltpu.VMEM((CHUNK, EMBED), jnp.float32),
            pltpu.SemaphoreType.DMA,
            pltpu.SemaphoreType.DMA,
        ],
        compiler_params=pltpu.CompilerParams(use_tc_tiling_on_sc=False),
    )
    def gather_k(idx_hbm, tab_hbm, out_hbm, idx_v, buf0, buf1, sem0, sem1):
        wid = lax.axis_index("s") * 2 + lax.axis_index("c")
        base = pl.multiple_of(wid * ROWS_PER_W, CHUNK)
        pltpu.sync_copy(idx_hbm.at[pl.ds(wid * NCHUNK, NCHUNK)], idx_v)

        pltpu.async_copy(tab_hbm.at[idx_v.at[0]], buf0, sem0)

        def body(g, _):
            j0 = g * 2
            pltpu.async_copy(tab_hbm.at[idx_v.at[j0 + 1]], buf1, sem1)
            pltpu.make_async_copy(tab_hbm.at[idx_v.at[j0]], buf0, sem0).wait()
            off0 = pl.multiple_of(base + j0 * CHUNK, CHUNK)
            pltpu.sync_copy(buf0, out_hbm.at[pl.ds(off0, CHUNK)])

            @pl.when(j0 + 2 < NCHUNK)
            def _start_even():
                pltpu.async_copy(tab_hbm.at[idx_v.at[j0 + 2]], buf0, sem0)

            pltpu.make_async_copy(tab_hbm.at[idx_v.at[j0 + 1]], buf1, sem1).wait()
            off1 = pl.multiple_of(base + (j0 + 1) * CHUNK, CHUNK)
            pltpu.sync_copy(buf1, out_hbm.at[pl.ds(off1, CHUNK)])
            return 0

        lax.fori_loop(0, NCHUNK // 2, body, 0)

    return gather_k


_sc_gather = _make_sc_gather()


def _mlp_body(x_ref, w1_ref, b1_ref, w2_ref, b2_ref, o_ref):
    x = x_ref[...]
    h = jnp.dot(x, w1_ref[...], preferred_element_type=jnp.float32) + b1_ref[...]
    h = h * jax.nn.sigmoid(h)
    o_ref[...] = jnp.dot(h, w2_ref[...], preferred_element_type=jnp.float32) + b2_ref[...]


def _mlp(x, w1t, b1, w2t, b2):
    bm = 2048
    grid = (BATCH // bm,)
    return pl.pallas_call(
        _mlp_body,
        grid=grid,
        in_specs=[
            pl.BlockSpec((bm, COND_DIM), lambda i: (i, 0)),
            pl.BlockSpec((COND_DIM, COND_DIM), lambda i: (0, 0)),
            pl.BlockSpec((1, COND_DIM), lambda i: (0, 0)),
            pl.BlockSpec((COND_DIM, COND_DIM), lambda i: (0, 0)),
            pl.BlockSpec((1, COND_DIM), lambda i: (0, 0)),
        ],
        out_specs=pl.BlockSpec((bm, COND_DIM), lambda i: (i, 0)),
        out_shape=jax.ShapeDtypeStruct((BATCH, COND_DIM), jnp.float32),
    )(x, w1t, b1, w2t, b2)


def kernel(condition, tables, W1, b1, W2, b2):
    flat_idx = (condition + jnp.arange(N_FIELDS, dtype=jnp.int32) * VOCAB).reshape(
        NW * NCHUNK, CHUNK
    )
    tab_flat = tables.reshape(N_FIELDS * VOCAB, EMBED)
    rows = _sc_gather(flat_idx, tab_flat)
    x = rows.reshape(BATCH, COND_DIM)
    return _mlp(x, W1.T, b1.reshape(1, COND_DIM), W2.T, b2.reshape(1, COND_DIM))


# speedup vs baseline: 18.3284x; 2.4357x over previous
import functools

import jax
import jax.numpy as jnp
from jax import lax
from jax.experimental import pallas as pl
from jax.experimental.pallas import tpu as pltpu
from jax.experimental.pallas import tpu_sc as plsc

N_FIELDS = 26
VOCAB = 100000
EMBED = 16
COND_DIM = N_FIELDS * EMBED
BATCH = 16384

NW = 32
ROWS_PER_W = COND_DIM // NW
BCH = 2048
NCH = BATCH // BCH
L = 16


def _make_sc_gather():
    mesh = plsc.VectorSubcoreMesh(core_axis_name="c", subcore_axis_name="s")

    @functools.partial(
        pl.kernel,
        mesh=mesh,
        out_type=jax.ShapeDtypeStruct((COND_DIM, BATCH), jnp.float32),
        scratch_types=[
            pltpu.VMEM((VOCAB,), jnp.float32),
            pltpu.VMEM((BATCH,), jnp.int32),
            pltpu.VMEM((BCH,), jnp.float32),
            pltpu.VMEM((BCH,), jnp.float32),
            pltpu.SemaphoreType.DMA,
            pltpu.SemaphoreType.DMA,
            pltpu.SemaphoreType.DMA,
        ],
        compiler_params=pltpu.CompilerParams(
            use_tc_tiling_on_sc=False, needs_layout_passes=False
        ),
    )
    def gather_k(ttab_hbm, condt_hbm, xt_hbm, row_v, idx_v, ob0, ob1, sem0, sem1, semr):
        wid = lax.axis_index("s") * 2 + lax.axis_index("c")
        r0 = wid * ROWS_PER_W

        obufs = (ob0, ob1)
        osems = (sem0, sem1)

        def do_row(r, _):
            f = r // EMBED
            @pl.when(jnp.logical_or(r == r0, lax.rem(r, EMBED) == 0))
            def _load_idx():
                pltpu.sync_copy(condt_hbm.at[f], idx_v)

            pltpu.sync_copy(ttab_hbm.at[r], row_v)

            def do_chunk(c, _):
                base = pl.multiple_of(c * BCH, BCH)
                slot = lax.rem(c, 2)

                def gather_into(ob):
                    for j in range(BCH // L):
                        idx = idx_v[pl.ds(base + j * L, L)]
                        ob[pl.ds(j * L, L)] = plsc.load_gather(row_v, [idx])

                @pl.when(c >= 2)
                def _drain0():
                    @pl.when(slot == 0)
                    def _():
                        pltpu.make_async_copy(ob0, xt_hbm.at[r, pl.ds(0, BCH)], sem0).wait()

                    @pl.when(slot == 1)
                    def _():
                        pltpu.make_async_copy(ob1, xt_hbm.at[r, pl.ds(0, BCH)], sem1).wait()

                @pl.when(slot == 0)
                def _g0():
                    gather_into(ob0)
                    pltpu.async_copy(ob0, xt_hbm.at[r, pl.ds(base, BCH)], sem0)

                @pl.when(slot == 1)
                def _g1():
                    gather_into(ob1)
                    pltpu.async_copy(ob1, xt_hbm.at[r, pl.ds(base, BCH)], sem1)

                return 0

            lax.fori_loop(0, NCH, do_chunk, 0)
            pltpu.make_async_copy(ob0, xt_hbm.at[r, pl.ds(0, BCH)], sem0).wait()
            pltpu.make_async_copy(ob1, xt_hbm.at[r, pl.ds(0, BCH)], sem1).wait()
            return 0

        lax.fori_loop(r0, r0 + ROWS_PER_W, do_row, 0)

    return gather_k


_sc_gather = _make_sc_gather()


def _mlp_body(xt_ref, w1_ref, b1_ref, w2_ref, b2_ref, ot_ref):
    xt = xt_ref[...]
    h = jnp.dot(w1_ref[...], xt, preferred_element_type=jnp.float32) + b1_ref[...]
    h = h * jax.nn.sigmoid(h)
    ot_ref[...] = jnp.dot(w2_ref[...], h, preferred_element_type=jnp.float32) + b2_ref[...]


def _mlp_t(xt, w1, b1, w2, b2):
    bn = 2048
    grid = (BATCH // bn,)
    return pl.pallas_call(
        _mlp_body,
        grid=grid,
        in_specs=[
            pl.BlockSpec((COND_DIM, bn), lambda i: (0, i)),
            pl.BlockSpec((COND_DIM, COND_DIM), lambda i: (0, 0)),
            pl.BlockSpec((COND_DIM, 1), lambda i: (0, 0)),
            pl.BlockSpec((COND_DIM, COND_DIM), lambda i: (0, 0)),
            pl.BlockSpec((COND_DIM, 1), lambda i: (0, 0)),
        ],
        out_specs=pl.BlockSpec((COND_DIM, bn), lambda i: (0, i)),
        out_shape=jax.ShapeDtypeStruct((COND_DIM, BATCH), jnp.float32),
    )(xt, w1, b1, w2, b2)


def kernel(condition, tables, W1, b1, W2, b2):
    ttab = tables.transpose(0, 2, 1).reshape(COND_DIM, VOCAB)
    condt = condition.T
    xt = _sc_gather(ttab, condt)
    ot = _mlp_t(xt, W1, b1.reshape(COND_DIM, 1), W2, b2.reshape(COND_DIM, 1))
    return ot.T


# speedup vs baseline: 38.7483x; 2.1141x over previous
import functools

import jax
import jax.numpy as jnp
from jax import lax
from jax.experimental import pallas as pl
from jax.experimental.pallas import tpu as pltpu
from jax.experimental.pallas import tpu_sc as plsc

N_FIELDS = 26
VOCAB = 100000
EMBED = 16
COND_DIM = N_FIELDS * EMBED
BATCH = 16384

NW = 32
ROWS_PER_W = COND_DIM // NW
BCH = 2048
NCH = BATCH // BCH
L = 16


def _make_sc_gather():
    mesh = plsc.VectorSubcoreMesh(core_axis_name="c", subcore_axis_name="s")

    @functools.partial(
        pl.kernel,
        mesh=mesh,
        out_type=jax.ShapeDtypeStruct((COND_DIM, BATCH), jnp.float32),
        scratch_types=[
            pltpu.VMEM((VOCAB,), jnp.float32),
            pltpu.VMEM((BATCH,), jnp.int32),
            pltpu.VMEM((BCH,), jnp.float32),
            pltpu.VMEM((BCH,), jnp.float32),
            pltpu.SemaphoreType.DMA,
            pltpu.SemaphoreType.DMA,
            pltpu.SemaphoreType.DMA,
        ],
        compiler_params=pltpu.CompilerParams(
            use_tc_tiling_on_sc=True, needs_layout_passes=False
        ),
    )
    def gather_k(ttab_hbm, condt_hbm, xt_hbm, row_v, idx_v, ob0, ob1, sem0, sem1, semr):
        wid = lax.axis_index("s") * 2 + lax.axis_index("c")
        r0 = wid * ROWS_PER_W

        obufs = (ob0, ob1)
        osems = (sem0, sem1)

        def do_row(r, _):
            f = r // EMBED
            @pl.when(jnp.logical_or(r == r0, lax.rem(r, EMBED) == 0))
            def _load_idx():
                pltpu.sync_copy(condt_hbm.at[f], idx_v)

            pltpu.sync_copy(ttab_hbm.at[r], row_v)

            def do_chunk(c, _):
                base = pl.multiple_of(c * BCH, BCH)
                slot = lax.rem(c, 2)

                def gather_into(ob):
                    for j in range(BCH // L):
                        idx = idx_v[pl.ds(base + j * L, L)]
                        ob[pl.ds(j * L, L)] = plsc.load_gather(row_v, [idx])

                @pl.when(c >= 2)
                def _drain0():
                    @pl.when(slot == 0)
                    def _():
                        pltpu.make_async_copy(ob0, xt_hbm.at[r, pl.ds(0, BCH)], sem0).wait()

                    @pl.when(slot == 1)
                    def _():
                        pltpu.make_async_copy(ob1, xt_hbm.at[r, pl.ds(0, BCH)], sem1).wait()

                @pl.when(slot == 0)
                def _g0():
                    gather_into(ob0)
                    pltpu.async_copy(ob0, xt_hbm.at[r, pl.ds(base, BCH)], sem0)

                @pl.when(slot == 1)
                def _g1():
                    gather_into(ob1)
                    pltpu.async_copy(ob1, xt_hbm.at[r, pl.ds(base, BCH)], sem1)

                return 0

            lax.fori_loop(0, NCH, do_chunk, 0)
            pltpu.make_async_copy(ob0, xt_hbm.at[r, pl.ds(0, BCH)], sem0).wait()
            pltpu.make_async_copy(ob1, xt_hbm.at[r, pl.ds(0, BCH)], sem1).wait()
            return 0

        lax.fori_loop(r0, r0 + ROWS_PER_W, do_row, 0)

    return gather_k


_sc_gather = _make_sc_gather()


def _mlp_body(xt_ref, w1_ref, b1_ref, w2_ref, b2_ref, ot_ref):
    xt = xt_ref[...]
    h = jnp.dot(w1_ref[...], xt, preferred_element_type=jnp.float32) + b1_ref[...]
    h = h * jax.nn.sigmoid(h)
    ot_ref[...] = jnp.dot(w2_ref[...], h, preferred_element_type=jnp.float32) + b2_ref[...]


def _mlp_t(xt, w1, b1, w2, b2):
    bn = 2048
    grid = (BATCH // bn,)
    return pl.pallas_call(
        _mlp_body,
        grid=grid,
        in_specs=[
            pl.BlockSpec((COND_DIM, bn), lambda i: (0, i)),
            pl.BlockSpec((COND_DIM, COND_DIM), lambda i: (0, 0)),
            pl.BlockSpec((COND_DIM, 1), lambda i: (0, 0)),
            pl.BlockSpec((COND_DIM, COND_DIM), lambda i: (0, 0)),
            pl.BlockSpec((COND_DIM, 1), lambda i: (0, 0)),
        ],
        out_specs=pl.BlockSpec((COND_DIM, bn), lambda i: (0, i)),
        out_shape=jax.ShapeDtypeStruct((COND_DIM, BATCH), jnp.float32),
    )(xt, w1, b1, w2, b2)


def kernel(condition, tables, W1, b1, W2, b2):
    ttab = tables.transpose(0, 2, 1).reshape(COND_DIM, VOCAB)
    condt = condition.T
    xt = _sc_gather(ttab, condt)
    ot = _mlp_t(xt, W1, b1.reshape(COND_DIM, 1), W2, b2.reshape(COND_DIM, 1))
    return ot.T


# speedup vs baseline: 47.4701x; 1.2251x over previous
import functools

import jax
import jax.numpy as jnp
from jax import lax
from jax.experimental import pallas as pl
from jax.experimental.pallas import tpu as pltpu
from jax.experimental.pallas import tpu_sc as plsc

N_FIELDS = 26
VOCAB = 100000
EMBED = 16
COND_DIM = N_FIELDS * EMBED
BATCH = 16384

NW = 32
ROWS_PER_W = COND_DIM // NW
BCH = 4096
NCH = BATCH // BCH
L = 16
QSTARTS = (0, 25088, 50176, 75264, 99968)
QLENS = (25088, 25088, 25088, 24704, 32)


def _make_sc_gather():
    mesh = plsc.VectorSubcoreMesh(core_axis_name="c", subcore_axis_name="s")

    @functools.partial(
        pl.kernel,
        mesh=mesh,
        out_type=jax.ShapeDtypeStruct((COND_DIM, BATCH), jnp.float32),
        scratch_types=[
            pltpu.VMEM((VOCAB,), jnp.float32),
            pltpu.VMEM((BATCH,), jnp.int32),
            pltpu.VMEM((BCH,), jnp.float32),
            pltpu.VMEM((BCH,), jnp.float32),
            pltpu.SemaphoreType.DMA,
            pltpu.SemaphoreType.DMA,
            pltpu.SemaphoreType.DMA,
        ],
        compiler_params=pltpu.CompilerParams(
            use_tc_tiling_on_sc=True, needs_layout_passes=False
        ),
    )
    def gather_k(ttab_hbm, condt_hbm, xt_hbm, row_v, idx_v, ob0, ob1, sem0, sem1, semr):
        wid = lax.axis_index("s") * 2 + lax.axis_index("c")
        r0 = wid * ROWS_PER_W

        obufs = (ob0, ob1)
        osems = (sem0, sem1)

        def do_row(r, _):
            f = r // EMBED
            @pl.when(jnp.logical_or(r == r0, lax.rem(r, EMBED) == 0))
            def _load_idx():
                pltpu.sync_copy(condt_hbm.at[f], idx_v)

            pltpu.sync_copy(ttab_hbm.at[r], row_v)

            for c in range(NCH):
                ob = obufs[c % 2]
                sem = osems[c % 2]
                if c >= 2:
                    pltpu.make_async_copy(ob, xt_hbm.at[r, pl.ds(0, BCH)], sem).wait()
                for j in range(BCH // L):
                    idx = idx_v[pl.ds(c * BCH + j * L, L)]
                    ob[pl.ds(j * L, L)] = plsc.load_gather(row_v, [idx])
                pltpu.async_copy(ob, xt_hbm.at[r, pl.ds(c * BCH, BCH)], sem)

            pltpu.make_async_copy(ob0, xt_hbm.at[r, pl.ds(0, BCH)], sem0).wait()
            pltpu.make_async_copy(ob1, xt_hbm.at[r, pl.ds(0, BCH)], sem1).wait()
            return 0

        lax.fori_loop(r0, r0 + ROWS_PER_W, do_row, 0)

    return gather_k


_sc_gather = _make_sc_gather()


def _mlp_body(xt_ref, w1_ref, b1_ref, w2_ref, b2_ref, ot_ref):
    xt = xt_ref[...]
    h = jnp.dot(w1_ref[...], xt, preferred_element_type=jnp.float32) + b1_ref[...]
    h = h * jax.nn.sigmoid(h)
    ot_ref[...] = jnp.dot(w2_ref[...], h, preferred_element_type=jnp.float32) + b2_ref[...]


def _mlp_t(xt, w1, b1, w2, b2):
    bn = 2048
    grid = (BATCH // bn,)
    return pl.pallas_call(
        _mlp_body,
        grid=grid,
        in_specs=[
            pl.BlockSpec((COND_DIM, bn), lambda i: (0, i)),
            pl.BlockSpec((COND_DIM, COND_DIM), lambda i: (0, 0)),
            pl.BlockSpec((COND_DIM, 1), lambda i: (0, 0)),
            pl.BlockSpec((COND_DIM, COND_DIM), lambda i: (0, 0)),
            pl.BlockSpec((COND_DIM, 1), lambda i: (0, 0)),
        ],
        out_specs=pl.BlockSpec((COND_DIM, bn), lambda i: (0, i)),
        out_shape=jax.ShapeDtypeStruct((COND_DIM, BATCH), jnp.float32),
    )(xt, w1, b1, w2, b2)


def kernel(condition, tables, W1, b1, W2, b2):
    ttab = tables.transpose(0, 2, 1).reshape(COND_DIM, VOCAB)
    condt = condition.T
    xt = _sc_gather(ttab, condt)
    ot = _mlp_t(xt, W1, b1.reshape(COND_DIM, 1), W2, b2.reshape(COND_DIM, 1))
    return ot.T


# speedup vs baseline: 67.3674x; 1.4192x over previous
import functools

import jax
import jax.numpy as jnp
from jax import lax
from jax.experimental import pallas as pl
from jax.experimental.pallas import tpu as pltpu
from jax.experimental.pallas import tpu_sc as plsc

N_FIELDS = 26
VOCAB = 100000
EMBED = 16
COND_DIM = N_FIELDS * EMBED
BATCH = 16384

NW = 32
ROWS_PER_W = COND_DIM // NW
BCH = 4096
NCH = BATCH // BCH
L = 16
QSTARTS = (0, 25088, 50176, 75264, 99968)
QLENS = (25088, 25088, 25088, 24704, 32)


def _make_sc_gather():
    mesh = plsc.VectorSubcoreMesh(core_axis_name="c", subcore_axis_name="s")

    @functools.partial(
        pl.kernel,
        mesh=mesh,
        out_type=jax.ShapeDtypeStruct((COND_DIM, BATCH), jnp.float32),
        scratch_types=[
            pltpu.VMEM((VOCAB,), jnp.float32),
            pltpu.VMEM((BATCH,), jnp.int32),
            pltpu.VMEM((BCH,), jnp.float32),
            pltpu.VMEM((BCH,), jnp.float32),
            pltpu.SemaphoreType.DMA,
            pltpu.SemaphoreType.DMA,
            pltpu.SemaphoreType.DMA,
        ],
        compiler_params=pltpu.CompilerParams(
            use_tc_tiling_on_sc=True,
            needs_layout_passes=False,
            disable_bounds_checks=True,
        ),
    )
    def gather_k(ttab_hbm, condt_hbm, xt_hbm, row_v, idx_v, ob0, ob1, sem0, sem1, semr):
        wid = lax.axis_index("s") * 2 + lax.axis_index("c")
        r0 = wid * ROWS_PER_W

        obufs = (ob0, ob1)
        osems = (sem0, sem1)

        def do_row(r, _):
            f = r // EMBED
            @pl.when(jnp.logical_or(r == r0, lax.rem(r, EMBED) == 0))
            def _load_idx():
                pltpu.sync_copy(condt_hbm.at[f], idx_v)

            pltpu.sync_copy(ttab_hbm.at[r], row_v)

            for c in range(NCH):
                ob = obufs[c % 2]
                sem = osems[c % 2]
                if c >= 2:
                    pltpu.make_async_copy(ob, xt_hbm.at[r, pl.ds(0, BCH)], sem).wait()

                @plsc.parallel_loop(0, BCH, L, unroll=8)
                def _gather(i):
                    idx = idx_v[pl.ds(c * BCH + i, L)]
                    ob[pl.ds(i, L)] = plsc.load_gather(row_v, [idx])

                pltpu.async_copy(ob, xt_hbm.at[r, pl.ds(c * BCH, BCH)], sem)

            pltpu.make_async_copy(ob0, xt_hbm.at[r, pl.ds(0, BCH)], sem0).wait()
            pltpu.make_async_copy(ob1, xt_hbm.at[r, pl.ds(0, BCH)], sem1).wait()
            return 0

        lax.fori_loop(r0, r0 + ROWS_PER_W, do_row, 0)

    return gather_k


_sc_gather = _make_sc_gather()


def _mlp_body(xt_ref, w1_ref, b1_ref, w2_ref, b2_ref, ot_ref):
    xt = xt_ref[...]
    h = jnp.dot(w1_ref[...], xt, preferred_element_type=jnp.float32) + b1_ref[...]
    h = h * jax.nn.sigmoid(h)
    ot_ref[...] = jnp.dot(w2_ref[...], h, preferred_element_type=jnp.float32) + b2_ref[...]


def _mlp_t(xt, w1, b1, w2, b2):
    bn = 2048
    grid = (BATCH // bn,)
    return pl.pallas_call(
        _mlp_body,
        grid=grid,
        in_specs=[
            pl.BlockSpec((COND_DIM, bn), lambda i: (0, i)),
            pl.BlockSpec((COND_DIM, COND_DIM), lambda i: (0, 0)),
            pl.BlockSpec((COND_DIM, 1), lambda i: (0, 0)),
            pl.BlockSpec((COND_DIM, COND_DIM), lambda i: (0, 0)),
            pl.BlockSpec((COND_DIM, 1), lambda i: (0, 0)),
        ],
        out_specs=pl.BlockSpec((COND_DIM, bn), lambda i: (0, i)),
        out_shape=jax.ShapeDtypeStruct((COND_DIM, BATCH), jnp.float32),
    )(xt, w1, b1, w2, b2)


def kernel(condition, tables, W1, b1, W2, b2):
    ttab = tables.transpose(0, 2, 1).reshape(COND_DIM, VOCAB)
    condt = condition.T
    xt = _sc_gather(ttab, condt)
    ot = _mlp_t(xt, W1, b1.reshape(COND_DIM, 1), W2, b2.reshape(COND_DIM, 1))
    return ot.T
